# baseline (speedup 1.0000x reference)
import jax
import jax.numpy as jnp
from jax.experimental import pallas as pl
from jax.experimental.pallas import tpu as pltpu

N = 4096
D = 128
PACK = 32
NW = N // PACK
TILE_A = 128
TILE_B = 512
NEG = float("-inf")


def _dott(a, b):
    return jax.lax.dot_general(a.astype(jnp.bfloat16), b.astype(jnp.bfloat16),
                               (((1,), (1,)), ((), ())),
                               preferred_element_type=jnp.float32)


def _bf(a):
    return a.astype(jnp.bfloat16).astype(jnp.float32)


def _pass1_body(adj21_ref, adj12_ref, f2_ref, f1_ref, x1_ref, x2_ref,
                w1s_ref, w1n_ref, w2s_ref, w2n_ref,
                s1_ref, s2_ref, g1_ref, g2_ref, b21_ref, b12_ref):
    adj21 = adj21_ref[...]
    adj12 = adj12_ref[...]
    f2 = f2_ref[...]
    f1 = f1_ref[...]

    m21 = adj21 > 0
    mx = jnp.max(jnp.where(m21, f2, NEG), axis=1, keepdims=True)
    s1 = jnp.where(mx == NEG, 0.0, mx)
    s2 = jnp.sum(jnp.where(adj12 > 0, f1, 0.0), axis=1, keepdims=True)
    s1_ref[...] = s1
    s2_ref[...] = s2

    c1 = jnp.sum(_bf(w1n_ref[0, :]))
    c2 = jnp.sum(_bf(w2n_ref[0, :]))
    a1 = jnp.sum(_bf(x1_ref[...]) * _bf(w1s_ref[0:1, :]), axis=1, keepdims=True)
    a2 = jnp.sum(_bf(x2_ref[...]) * _bf(w2s_ref[0:1, :]), axis=1, keepdims=True)
    g1_ref[...] = jnp.maximum(a1 + _bf(s1) * c1, 0.0)
    g2_ref[...] = jnp.maximum(a2 + _bf(s2) * c2, 0.0)

    b21 = jnp.zeros((adj21.shape[0], NW), dtype=jnp.uint32)
    b12 = jnp.zeros((adj21.shape[0], NW), dtype=jnp.uint32)
    for k in range(PACK):
        sl = slice(k * NW, (k + 1) * NW)
        b21 = b21 + (m21[:, sl].astype(jnp.uint32) << k)
        b12 = b12 + ((adj12[:, sl] > 0).astype(jnp.uint32) << k)
    b21_ref[...] = b21
    b12_ref[...] = b12


def _pass2_body(b21_ref, b12_ref, g2r_ref, g1r_ref, x1_ref, x2_ref,
                s1_ref, s2_ref,
                w1s0_ref, w1n0_ref, w2s0_ref, w2n0_ref,
                w1s1_ref, w1n1_ref, w2s1_ref, w2n1_ref,
                o1_ref, o2_ref):
    b21 = b21_ref[...]
    b12 = b12_ref[...]
    t = b21.shape[0]

    mx = jnp.full((t, NW), NEG, dtype=jnp.float32)
    acc = jnp.zeros((t, NW), dtype=jnp.float32)
    for k in range(PACK):
        sl = slice(k * NW, (k + 1) * NW)
        g2k = g2r_ref[0:1, sl]
        g1k = g1r_ref[0:1, sl]
        m21k = ((b21 >> k) & 1) > 0
        m12k = ((b12 >> k) & 1).astype(jnp.float32)
        mx = jnp.maximum(mx, jnp.where(m21k, g2k, NEG))
        acc = acc + m12k * g1k
    mxr = jnp.max(mx, axis=1, keepdims=True)
    s1p = jnp.where(mxr == NEG, 0.0, mxr)
    s2p = jnp.sum(acc, axis=1, keepdims=True)

    r1n0 = jnp.sum(_bf(w1n0_ref[...]), axis=1)[None, :]
    r2n0 = jnp.sum(_bf(w2n0_ref[...]), axis=1)[None, :]
    r1n1 = jnp.sum(_bf(w1n1_ref[...]), axis=1)[None, :]
    r2n1 = jnp.sum(_bf(w2n1_ref[...]), axis=1)[None, :]

    h1 = jnp.maximum(_dott(x1_ref[...], w1s0_ref[...]) + _bf(s1_ref[...]) * r1n0, 0.0)
    h2 = jnp.maximum(_dott(x2_ref[...], w2s0_ref[...]) + _bf(s2_ref[...]) * r2n0, 0.0)
    o1_ref[...] = _dott(h1, w1s1_ref[...]) + _bf(s1p) * r1n1
    o2_ref[...] = _dott(h2, w2s1_ref[...]) + _bf(s2p) * r2n1


def kernel(x1, x2, adj_1to2, adj_2to1,
           l0_w1_self, l0_w1_neigh, l0_w2_self, l0_w2_neigh,
           l1_w1_self, l1_w1_neigh, l1_w2_self, l1_w2_neigh):
    f2 = x2[:, 0].reshape(1, N)
    f1 = x1[:, 0].reshape(1, N)

    row_t = lambda i: (i, 0)
    full = lambda i: (0, 0)
    grid_a = (N // TILE_A,)
    s1, s2, g1, g2, b21, b12 = pl.pallas_call(
        _pass1_body,
        grid=grid_a,
        in_specs=[
            pl.BlockSpec((TILE_A, N), row_t),
            pl.BlockSpec((TILE_A, N), row_t),
            pl.BlockSpec((1, N), full),
            pl.BlockSpec((1, N), full),
            pl.BlockSpec((TILE_A, D), row_t),
            pl.BlockSpec((TILE_A, D), row_t),
            pl.BlockSpec((D, D), full),
            pl.BlockSpec((D, D), full),
            pl.BlockSpec((D, D), full),
            pl.BlockSpec((D, D), full),
        ],
        out_specs=[
            pl.BlockSpec((TILE_A, 1), row_t),
            pl.BlockSpec((TILE_A, 1), row_t),
            pl.BlockSpec((TILE_A, 1), row_t),
            pl.BlockSpec((TILE_A, 1), row_t),
            pl.BlockSpec((TILE_A, NW), row_t),
            pl.BlockSpec((TILE_A, NW), row_t),
        ],
        out_shape=[
            jax.ShapeDtypeStruct((N, 1), jnp.float32),
            jax.ShapeDtypeStruct((N, 1), jnp.float32),
            jax.ShapeDtypeStruct((N, 1), jnp.float32),
            jax.ShapeDtypeStruct((N, 1), jnp.float32),
            jax.ShapeDtypeStruct((N, NW), jnp.uint32),
            jax.ShapeDtypeStruct((N, NW), jnp.uint32),
        ],
        compiler_params=pltpu.CompilerParams(
            dimension_semantics=("arbitrary",)),
    )(adj_2to1, adj_1to2, f2, f1, x1, x2,
      l0_w1_self, l0_w1_neigh, l0_w2_self, l0_w2_neigh)

    g1r = g1.reshape(1, N)
    g2r = g2.reshape(1, N)

    grid_b = (N // TILE_B,)
    o1, o2 = pl.pallas_call(
        _pass2_body,
        grid=grid_b,
        in_specs=[
            pl.BlockSpec((TILE_B, NW), row_t),
            pl.BlockSpec((TILE_B, NW), row_t),
            pl.BlockSpec((1, N), full),
            pl.BlockSpec((1, N), full),
            pl.BlockSpec((TILE_B, D), row_t),
            pl.BlockSpec((TILE_B, D), row_t),
            pl.BlockSpec((TILE_B, 1), row_t),
            pl.BlockSpec((TILE_B, 1), row_t),
            pl.BlockSpec((D, D), full),
            pl.BlockSpec((D, D), full),
            pl.BlockSpec((D, D), full),
            pl.BlockSpec((D, D), full),
            pl.BlockSpec((D, D), full),
            pl.BlockSpec((D, D), full),
            pl.BlockSpec((D, D), full),
            pl.BlockSpec((D, D), full),
        ],
        out_specs=[
            pl.BlockSpec((TILE_B, D), row_t),
            pl.BlockSpec((TILE_B, D), row_t),
        ],
        out_shape=[
            jax.ShapeDtypeStruct((N, D), jnp.float32),
            jax.ShapeDtypeStruct((N, D), jnp.float32),
        ],
        compiler_params=pltpu.CompilerParams(
            dimension_semantics=("arbitrary",)),
    )(b21, b12, g2r, g1r, x1, x2, s1, s2,
      l0_w1_self, l0_w1_neigh, l0_w2_self, l0_w2_neigh,
      l1_w1_self, l1_w1_neigh, l1_w2_self, l1_w2_neigh)

    return (o1, o2)
